# baseline (speedup 1.0000x reference)
import jax
import jax.numpy as jnp
import numpy as np
from jax.experimental import pallas as pl
from jax.experimental.pallas import tpu as pltpu

_KS = (3, 5, 5, 5, 3, 7, 8, 3, 3, 3)
_OFFS = tuple(int(o) for o in np.concatenate([[0], np.cumsum(_KS)[:-1]]))
_KTOT = int(sum(_KS))
_ROWS_PER_BLOCK = 8192


def _vq_body(x_ref, gt_ref, adjct_ref, rel_ref, pt_ref, b2t_ref,
             o0_ref, o1_ref):
    nb, nj, v = x_ref.shape
    r = nb * nj
    x = x_ref[...].reshape(r, v)
    xt = x.T
    st = jax.lax.dot_general(
        gt_ref[...], xt.astype(jnp.bfloat16), (((1,), (0,)), ((), ())),
        preferred_element_type=jnp.float32,
    )
    adjt = adjct_ref[...] - 2.0 * st
    b = jax.lax.bitcast_convert_type(adjt, jnp.int32)
    key = b ^ ((b >> 31) & jnp.int32(0x7FFFFFFF))
    key = (key & jnp.int32(-8)) | rel_ref[...]
    ohs = []
    for i in range(10):
        o, k = _OFFS[i], _KS[i]
        blk = key[o:o + k, :]
        m = jnp.min(blk, axis=0, keepdims=True)
        ohs.append((blk == m).astype(jnp.bfloat16))
    oht = jnp.concatenate(ohs, axis=0)
    qt = jax.lax.dot_general(
        pt_ref[...], oht, (((1,), (0,)), ((), ())),
        preferred_element_type=jnp.float32,
    ) + b2t_ref[...]
    o0_ref[...] = qt.T.reshape(nb, nj, v)
    dt = xt - qt
    dsq = (dt * dt).astype(jnp.bfloat16)
    ones = jnp.ones((1, 64), dtype=jnp.bfloat16)
    o1 = jax.lax.dot_general(
        ones, dsq, (((1,), (0,)), ((), ())),
        preferred_element_type=jnp.float32,
    )
    o1_ref[...] = o1.reshape(nb, nj)


def kernel(x0, W1, b1, cb0, cb1, cb2, cb3, cb4, cb5, cb6, cb7, cb8, cb9,
           W2, b2):
    cbs = (cb0, cb1, cb2, cb3, cb4, cb5, cb6, cb7, cb8, cb9)
    B0, B1, V = x0.shape

    g_cols, sb_parts, cbn_parts, p_rows = [], [], [], []
    for i, cb in enumerate(cbs):
        c = cb[0]
        w1s = W1[16 * i:16 * (i + 1), :]
        g_cols.append(w1s.T @ c.T)
        sb_parts.append(c @ b1[16 * i:16 * (i + 1)])
        cbn_parts.append(jnp.sum(c * c, axis=1))
        p_rows.append(c @ W2[:, i::10].T)
    Gt = jnp.concatenate(g_cols, axis=1).T.astype(jnp.bfloat16)
    adjct = (jnp.concatenate(cbn_parts)
             - 2.0 * jnp.concatenate(sb_parts))[:, None]
    rel = np.concatenate([np.arange(k, dtype=np.int32) for k in _KS])[:, None]
    Pt = jnp.concatenate(p_rows, axis=0).T.astype(jnp.bfloat16)
    b2t = b2[:, None]

    NB = 8
    J = _ROWS_PER_BLOCK // NB
    grid = (B0 // NB, B1 // J)
    out0, out1 = pl.pallas_call(
        _vq_body,
        grid=grid,
        in_specs=[
            pl.BlockSpec((NB, J, V), lambda i, j: (i, j, 0)),
            pl.BlockSpec((_KTOT, V), lambda i, j: (0, 0)),
            pl.BlockSpec((_KTOT, 1), lambda i, j: (0, 0)),
            pl.BlockSpec((_KTOT, 1), lambda i, j: (0, 0)),
            pl.BlockSpec((V, _KTOT), lambda i, j: (0, 0)),
            pl.BlockSpec((V, 1), lambda i, j: (0, 0)),
        ],
        out_specs=[
            pl.BlockSpec((NB, J, V), lambda i, j: (i, j, 0)),
            pl.BlockSpec((NB, J), lambda i, j: (i, j)),
        ],
        out_shape=[
            jax.ShapeDtypeStruct((B0, B1, V), jnp.float32),
            jax.ShapeDtypeStruct((B0, B1), jnp.float32),
        ],
        compiler_params=pltpu.CompilerParams(
            dimension_semantics=("arbitrary", "arbitrary"),
        ),
    )(x0, Gt, adjct, jnp.asarray(rel), Pt, b2t)

    entropy = jnp.asarray(0.0, dtype=jnp.float32)
    return (out0, out1, out1, entropy)


# speedup vs baseline: 1.0678x; 1.0678x over previous
import jax
import jax.numpy as jnp
import numpy as np
from jax.experimental import pallas as pl
from jax.experimental.pallas import tpu as pltpu

_KS = (3, 5, 5, 5, 3, 7, 8, 3, 3, 3)
_KMAX = 8
_NBOOK = 10
_KTOT = _KMAX * _NBOOK
_ROWS_PER_BLOCK = 8192


def _vq_body(x_ref, gt_ref, adjct_ref, pt_ref, b2t_ref, o0_ref, o1_ref):
    nb, nj, v = x_ref.shape
    r = nb * nj
    x = x_ref[...].reshape(r, v)
    xt = x.T
    st = jax.lax.dot_general(
        gt_ref[...], xt.astype(jnp.bfloat16), (((1,), (0,)), ((), ())),
        preferred_element_type=jnp.float32,
    )
    adjt = adjct_ref[...] - 2.0 * st
    b = jax.lax.bitcast_convert_type(adjt, jnp.int32)
    key = b ^ ((b >> 31) & jnp.int32(0x7FFFFFFF))
    rel = jax.lax.broadcasted_iota(jnp.int32, (_KTOT, 1), 0) & jnp.int32(7)
    key = (key & jnp.int32(-8)) | rel
    ohs = []
    for i in range(_NBOOK):
        o = _KMAX * i
        blk = key[o:o + _KMAX, :]
        m = jnp.min(blk, axis=0, keepdims=True)
        ohs.append((blk == m).astype(jnp.bfloat16))
    oht = jnp.concatenate(ohs, axis=0)
    qt = jax.lax.dot_general(
        pt_ref[...], oht, (((1,), (0,)), ((), ())),
        preferred_element_type=jnp.float32,
    ) + b2t_ref[...]
    o0_ref[...] = qt.T.reshape(nb, nj, v)
    dt = xt - qt
    dsq = (dt * dt).astype(jnp.bfloat16)
    ones = jnp.ones((1, 64), dtype=jnp.bfloat16)
    o1 = jax.lax.dot_general(
        ones, dsq, (((1,), (0,)), ((), ())),
        preferred_element_type=jnp.float32,
    )
    o1_ref[...] = o1.reshape(nb, nj)


def kernel(x0, W1, b1, cb0, cb1, cb2, cb3, cb4, cb5, cb6, cb7, cb8, cb9,
           W2, b2):
    cbs = (cb0, cb1, cb2, cb3, cb4, cb5, cb6, cb7, cb8, cb9)
    B0, B1, V = x0.shape

    cbp = jnp.stack([
        jnp.pad(cb[0], ((0, _KMAX - cb.shape[1]), (0, 0))) for cb in cbs
    ])
    w1r = W1.reshape(_NBOOK, 16, V)
    b1r = b1.reshape(_NBOOK, 16)
    w2r = W2.T.reshape(16, _NBOOK, V).transpose(1, 0, 2)
    Gt = jnp.einsum("bke,bed->bkd", cbp, w1r).reshape(_KTOT, V)
    Gt = Gt.astype(jnp.bfloat16)
    sb = jnp.einsum("bke,be->bk", cbp, b1r)
    cbn = jnp.sum(cbp * cbp, axis=2)
    pad = np.zeros((_NBOOK, _KMAX), np.float32)
    for i, k in enumerate(_KS):
        pad[i, k:] = 1e30
    adjct = (cbn - 2.0 * sb + pad).reshape(_KTOT, 1)
    Pt = jnp.einsum("bke,bed->bkd", cbp, w2r).reshape(_KTOT, V).T
    Pt = Pt.astype(jnp.bfloat16)
    b2t = b2[:, None]

    NB = 8
    J = _ROWS_PER_BLOCK // NB
    grid = (B0 // NB, B1 // J)
    out0, out1 = pl.pallas_call(
        _vq_body,
        grid=grid,
        in_specs=[
            pl.BlockSpec((NB, J, V), lambda i, j: (i, j, 0)),
            pl.BlockSpec((_KTOT, V), lambda i, j: (0, 0)),
            pl.BlockSpec((_KTOT, 1), lambda i, j: (0, 0)),
            pl.BlockSpec((V, _KTOT), lambda i, j: (0, 0)),
            pl.BlockSpec((V, 1), lambda i, j: (0, 0)),
        ],
        out_specs=[
            pl.BlockSpec((NB, J, V), lambda i, j: (i, j, 0)),
            pl.BlockSpec((NB, J), lambda i, j: (i, j)),
        ],
        out_shape=[
            jax.ShapeDtypeStruct((B0, B1, V), jnp.float32),
            jax.ShapeDtypeStruct((B0, B1), jnp.float32),
        ],
        compiler_params=pltpu.CompilerParams(
            dimension_semantics=("arbitrary", "arbitrary"),
        ),
    )(x0, Gt, adjct, Pt, b2t)

    entropy = jnp.asarray(0.0, dtype=jnp.float32)
    return (out0, out1, out1, entropy)


# speedup vs baseline: 1.0995x; 1.0297x over previous
import jax
import jax.numpy as jnp
import numpy as np
from jax.experimental import pallas as pl
from jax.experimental.pallas import tpu as pltpu

_KS = (3, 5, 5, 5, 3, 7, 8, 3, 3, 3)
_KMAX = 8
_NBOOK = 10
_KTOT = _KMAX * _NBOOK
_ROWS_PER_BLOCK = 16384


def _vq_body(x_ref, gt_ref, adjct_ref, pt_ref, b2t_ref, o0_ref, o1_ref,
             o2_ref):
    nb, nj, v = x_ref.shape
    r = nb * nj
    x = x_ref[...].reshape(r, v)
    xt = x.T
    st = jax.lax.dot_general(
        gt_ref[...], xt.astype(jnp.bfloat16), (((1,), (0,)), ((), ())),
        preferred_element_type=jnp.float32,
    )
    adjt = adjct_ref[...] - 2.0 * st
    b = jax.lax.bitcast_convert_type(adjt, jnp.int32)
    key = b ^ ((b >> 31) & jnp.int32(0x7FFFFFFF))
    rel = jax.lax.broadcasted_iota(jnp.int32, (_KTOT, 1), 0) & jnp.int32(7)
    key = (key & jnp.int32(-8)) | rel
    ohs = []
    for i in range(_NBOOK):
        o = _KMAX * i
        blk = key[o:o + _KMAX, :]
        m = jnp.min(blk, axis=0, keepdims=True)
        ohs.append((blk == m).astype(jnp.bfloat16))
    oht = jnp.concatenate(ohs, axis=0)
    qt = jax.lax.dot_general(
        pt_ref[...], oht, (((1,), (0,)), ((), ())),
        preferred_element_type=jnp.float32,
    ) + b2t_ref[...]
    o0_ref[...] = qt.T.reshape(nb, nj, v)
    dt = xt - qt
    dsq = (dt * dt).astype(jnp.bfloat16)
    ones = jnp.ones((1, 64), dtype=jnp.bfloat16)
    o1 = jax.lax.dot_general(
        ones, dsq, (((1,), (0,)), ((), ())),
        preferred_element_type=jnp.float32,
    ).reshape(nb, nj)
    o1_ref[...] = o1
    o2_ref[...] = o1


def kernel(x0, W1, b1, cb0, cb1, cb2, cb3, cb4, cb5, cb6, cb7, cb8, cb9,
           W2, b2):
    cbs = (cb0, cb1, cb2, cb3, cb4, cb5, cb6, cb7, cb8, cb9)
    B0, B1, V = x0.shape

    cbp = jnp.stack([
        jnp.pad(cb[0], ((0, _KMAX - cb.shape[1]), (0, 0))) for cb in cbs
    ])
    w1r = W1.reshape(_NBOOK, 16, V)
    b1r = b1.reshape(_NBOOK, 16)
    w2r = W2.T.reshape(16, _NBOOK, V).transpose(1, 0, 2)
    Gt = jnp.einsum("bke,bed->bkd", cbp, w1r).reshape(_KTOT, V)
    Gt = Gt.astype(jnp.bfloat16)
    sb = jnp.einsum("bke,be->bk", cbp, b1r)
    cbn = jnp.sum(cbp * cbp, axis=2)
    pad = np.zeros((_NBOOK, _KMAX), np.float32)
    for i, k in enumerate(_KS):
        pad[i, k:] = 1e30
    adjct = (cbn - 2.0 * sb + pad).reshape(_KTOT, 1)
    Pt = jnp.einsum("bke,bed->bkd", cbp, w2r).reshape(_KTOT, V).T
    Pt = Pt.astype(jnp.bfloat16)
    b2t = b2[:, None]

    return _launch(x0, Gt, adjct, Pt, b2t)


def _launch(x0, Gt, adjct, Pt, b2t):
    B0, B1, V = x0.shape
    NB = 8
    J = _ROWS_PER_BLOCK // NB
    grid = (B0 // NB, B1 // J)
    out0, out1, out2 = pl.pallas_call(
        _vq_body,
        grid=grid,
        in_specs=[
            pl.BlockSpec((NB, J, V), lambda i, j: (i, j, 0)),
            pl.BlockSpec((_KTOT, V), lambda i, j: (0, 0)),
            pl.BlockSpec((_KTOT, 1), lambda i, j: (0, 0)),
            pl.BlockSpec((V, _KTOT), lambda i, j: (0, 0)),
            pl.BlockSpec((V, 1), lambda i, j: (0, 0)),
        ],
        out_specs=[
            pl.BlockSpec((NB, J, V), lambda i, j: (i, j, 0)),
            pl.BlockSpec((NB, J), lambda i, j: (i, j)),
            pl.BlockSpec((NB, J), lambda i, j: (i, j)),
        ],
        out_shape=[
            jax.ShapeDtypeStruct((B0, B1, V), jnp.float32),
            jax.ShapeDtypeStruct((B0, B1), jnp.float32),
            jax.ShapeDtypeStruct((B0, B1), jnp.float32),
        ],
        compiler_params=pltpu.CompilerParams(
            dimension_semantics=("arbitrary", "arbitrary"),
        ),
    )(x0, Gt, adjct, Pt, b2t)

    entropy = jnp.asarray(0.0, dtype=jnp.float32)
    return (out0, out1, out2, entropy)


# speedup vs baseline: 1.1196x; 1.0182x over previous
import jax
import jax.numpy as jnp
import numpy as np
from jax.experimental import pallas as pl
from jax.experimental.pallas import tpu as pltpu

_KS = (3, 5, 5, 5, 3, 7, 8, 3, 3, 3)
_KMAX = 8
_NBOOK = 10
_KTOT = _KMAX * _NBOOK
_ROWS_PER_BLOCK = 16384
_H = 16 * _NBOOK


def _vq_body(x_ref, w1_ref, b1_ref, w2_ref, b2t_ref,
             cb0_ref, cb1_ref, cb2_ref, cb3_ref, cb4_ref,
             cb5_ref, cb6_ref, cb7_ref, cb8_ref, cb9_ref,
             o0_ref, o1_ref, o2_ref, gt_s, adjct_s, pt_s):
    cb_refs = (cb0_ref, cb1_ref, cb2_ref, cb3_ref, cb4_ref,
               cb5_ref, cb6_ref, cb7_ref, cb8_ref, cb9_ref)

    @pl.when((pl.program_id(0) == 0) & (pl.program_id(1) == 0))
    def _fold():
        f32 = jnp.float32
        w1 = w1_ref[...]
        w2 = w2_ref[...]
        gts, adjs, pts = [], [], []
        for i in range(_NBOOK):
            k = _KS[i]
            c = cb_refs[i][0]
            w1s = w1[16 * i:16 * (i + 1), :]
            gt_i = jax.lax.dot_general(
                c, w1s, (((1,), (0,)), ((), ())), preferred_element_type=f32)
            gts.append(jnp.pad(gt_i, ((0, _KMAX - k), (0, 0))))
            b1s = b1_ref[0:1, 16 * i:16 * (i + 1)]
            sb_i = jax.lax.dot_general(
                c, b1s, (((1,), (1,)), ((), ())), preferred_element_type=f32)
            cbn_i = jnp.sum(c * c, axis=1, keepdims=True)
            adjs.append(jnp.pad(cbn_i - 2.0 * sb_i, ((0, _KMAX - k), (0, 0)),
                                constant_values=1e30))
            ie = jax.lax.broadcasted_iota(jnp.int32, (_H, 16), 0)
            idd = jax.lax.broadcasted_iota(jnp.int32, (_H, 16), 1)
            sel = (ie == idd * _NBOOK + i).astype(f32)
            w2s = jax.lax.dot_general(
                w2, sel, (((1,), (0,)), ((), ())), preferred_element_type=f32)
            p_i = jax.lax.dot_general(
                c, w2s, (((1,), (1,)), ((), ())), preferred_element_type=f32)
            pts.append(jnp.pad(p_i, ((0, _KMAX - k), (0, 0))))
        gt_s[...] = jnp.concatenate(gts, axis=0).astype(jnp.bfloat16)
        adjct_s[...] = jnp.concatenate(adjs, axis=0)
        pt_s[...] = jnp.concatenate(pts, axis=0).T.astype(jnp.bfloat16)

    nb, nj, v = x_ref.shape
    r = nb * nj
    x = x_ref[...].reshape(r, v)
    xt = x.T
    st = jax.lax.dot_general(
        gt_s[...], xt.astype(jnp.bfloat16), (((1,), (0,)), ((), ())),
        preferred_element_type=jnp.float32,
    )
    adjt = adjct_s[...] - 2.0 * st
    b = jax.lax.bitcast_convert_type(adjt, jnp.int32)
    key = b ^ ((b >> 31) & jnp.int32(0x7FFFFFFF))
    rel = jax.lax.broadcasted_iota(jnp.int32, (_KTOT, 1), 0) & jnp.int32(7)
    key = (key & jnp.int32(-8)) | rel
    ohs = []
    for i in range(_NBOOK):
        o = _KMAX * i
        blk = key[o:o + _KMAX, :]
        m = jnp.min(blk, axis=0, keepdims=True)
        ohs.append((blk == m).astype(jnp.bfloat16))
    oht = jnp.concatenate(ohs, axis=0)
    qt = jax.lax.dot_general(
        pt_s[...], oht, (((1,), (0,)), ((), ())),
        preferred_element_type=jnp.float32,
    ) + b2t_ref[...]
    o0_ref[...] = qt.T.reshape(nb, nj, v)
    dt = xt - qt
    dsq = (dt * dt).astype(jnp.bfloat16)
    ones = jnp.ones((1, 64), dtype=jnp.bfloat16)
    o1 = jax.lax.dot_general(
        ones, dsq, (((1,), (0,)), ((), ())),
        preferred_element_type=jnp.float32,
    ).reshape(nb, nj)
    o1_ref[...] = o1
    o2_ref[...] = o1


def kernel(x0, W1, b1, cb0, cb1, cb2, cb3, cb4, cb5, cb6, cb7, cb8, cb9,
           W2, b2):
    cbs = (cb0, cb1, cb2, cb3, cb4, cb5, cb6, cb7, cb8, cb9)
    B0, B1, V = x0.shape

    NB = 8
    J = _ROWS_PER_BLOCK // NB
    grid = (B0 // NB, B1 // J)
    const = lambda i, j: (0, 0)
    cb_specs = [
        pl.BlockSpec((1, cb.shape[1], 16), lambda i, j: (0, 0, 0))
        for cb in cbs
    ]
    out0, out1, out2 = pl.pallas_call(
        _vq_body,
        grid=grid,
        in_specs=[
            pl.BlockSpec((NB, J, V), lambda i, j: (i, j, 0)),
            pl.BlockSpec((_H, V), const),
            pl.BlockSpec((1, _H), const),
            pl.BlockSpec((V, _H), const),
            pl.BlockSpec((V, 1), const),
        ] + cb_specs,
        out_specs=[
            pl.BlockSpec((NB, J, V), lambda i, j: (i, j, 0)),
            pl.BlockSpec((NB, J), lambda i, j: (i, j)),
            pl.BlockSpec((NB, J), lambda i, j: (i, j)),
        ],
        out_shape=[
            jax.ShapeDtypeStruct((B0, B1, V), jnp.float32),
            jax.ShapeDtypeStruct((B0, B1), jnp.float32),
            jax.ShapeDtypeStruct((B0, B1), jnp.float32),
        ],
        scratch_shapes=[
            pltpu.VMEM((_KTOT, V), jnp.bfloat16),
            pltpu.VMEM((_KTOT, 1), jnp.float32),
            pltpu.VMEM((V, _KTOT), jnp.bfloat16),
        ],
        compiler_params=pltpu.CompilerParams(
            dimension_semantics=("arbitrary", "arbitrary"),
        ),
    )(x0, W1, b1[None, :], W2, b2[:, None], *cbs)

    entropy = jnp.asarray(0.0, dtype=jnp.float32)
    return (out0, out1, out2, entropy)
